# baseline (speedup 1.0000x reference)
import functools

import jax
import jax.numpy as jnp
from jax import lax
from jax.experimental import pallas as pl
from jax.experimental.pallas import tpu as pltpu
from jax.experimental.pallas import tpu_sc as plsc

B = 16384
EMB = 32
IDX_CHUNK = 128

NU = 1000000
NM = 100000
U_FULL = (NU // 128) * 128
M_FULL = (NM // 128) * 128
CH = 768
CHR = CH // 4
U_CHUNKS = U_FULL // CH
M_CHUNKS = (M_FULL - 128) // CH
M_LEFT = M_CHUNKS * CH


def _sc_mesh():
    info = plsc.get_sparse_core_info()
    return (plsc.VectorSubcoreMesh(core_axis_name="c", subcore_axis_name="s"),
            info.num_cores, info.num_subcores)


def _sc_transpose_pack(utT, mtT):
    mesh, nc, ns = _sc_mesh()
    nw = nc * ns

    kpw_u = (U_CHUNKS + nw - 1) // nw
    kpw_m = (M_CHUNKS + nw - 1) // nw

    @functools.partial(
        pl.kernel,
        mesh=mesh,
        out_type=(
            jax.ShapeDtypeStruct((NU // 4, 128), jnp.float32),
            jax.ShapeDtypeStruct((NM // 4, 128), jnp.float32),
        ),
        scratch_types=[
            pltpu.VMEM((EMB, CH), jnp.float32),
            pltpu.VMEM((CHR, 128), jnp.float32),
        ],
        compiler_params=pltpu.CompilerParams(
            use_tc_tiling_on_sc=True, needs_layout_passes=False),
    )
    def tp_kernel(ut, mt, up, mp, inbuf, outbuf):
        wid = lax.axis_index("s") * nc + lax.axis_index("c")
        rows_lo = lax.iota(jnp.int32, 16)
        rows_hi = rows_lo + 16

        def transpose_rows(nrows):
            @pl.loop(0, nrows, unroll=8)
            def _(r):
                for k4 in range(4):
                    col = rows_lo * 0 + (r * 4 + k4)
                    lo = plsc.load_gather(inbuf, [rows_lo, col])
                    hi = plsc.load_gather(inbuf, [rows_hi, col])
                    outbuf[r, pl.ds(k4 * 32, 16)] = lo
                    outbuf[r, pl.ds(k4 * 32 + 16, 16)] = hi

        def do_chunks(src, dst, nchunks, kpw):
            @pl.loop(0, kpw)
            def _(k):
                cid = wid + k * nw

                @pl.when(cid < nchunks)
                def _():
                    pltpu.sync_copy(src.at[:, pl.ds(cid * CH, CH)], inbuf)
                    transpose_rows(CHR)
                    pltpu.sync_copy(outbuf, dst.at[pl.ds(cid * CHR, CHR)])

        do_chunks(ut, up, U_CHUNKS, kpw_u)
        do_chunks(mt, mp, M_CHUNKS, kpw_m)

        @pl.when(wid == nw - 1)
        def _():
            pltpu.sync_copy(mt.at[:, pl.ds(M_LEFT, 128)],
                            inbuf.at[:, pl.ds(0, 128)])
            transpose_rows(32)
            pltpu.sync_copy(outbuf.at[pl.ds(0, 32)],
                            mp.at[pl.ds(M_LEFT // 4, 32)])

    return tp_kernel(utT, mtT)


def _sc_gather(ut2, mt2, uj2, mj2):
    mesh, nc, ns = _sc_mesh()
    nw = nc * ns
    bpw = B // nw
    cpw = bpw // IDX_CHUNK

    @functools.partial(
        pl.kernel,
        mesh=mesh,
        out_type=(
            jax.ShapeDtypeStruct((B, 128), jnp.float32),
            jax.ShapeDtypeStruct((B, 128), jnp.float32),
        ),
        scratch_types=[
            pltpu.VMEM((cpw, IDX_CHUNK), jnp.int32),
            pltpu.VMEM((cpw, IDX_CHUNK), jnp.int32),
            pltpu.VMEM((bpw, 128), jnp.float32),
            pltpu.SemaphoreType.DMA,
        ],
        compiler_params=pltpu.CompilerParams(use_tc_tiling_on_sc=True),
    )
    def gather_kernel(ut_hbm, mt_hbm, uidx_hbm, midx_hbm, u_out, m_out,
                      uidx_v, midx_v, rows_v, sem):
        wid = lax.axis_index("s") * nc + lax.axis_index("c")
        base = wid * bpw
        pltpu.sync_copy(uidx_hbm.at[pl.ds(wid * cpw, cpw)], uidx_v)
        pltpu.sync_copy(midx_hbm.at[pl.ds(wid * cpw, cpw)], midx_v)
        copies = []
        for j in range(cpw):
            copies.append(pltpu.async_copy(
                ut_hbm.at[uidx_v.at[j]],
                rows_v.at[pl.ds(j * IDX_CHUNK, IDX_CHUNK)], sem))
        for c in copies:
            c.wait()
        pltpu.sync_copy(rows_v, u_out.at[pl.ds(base, bpw)])
        copies = []
        for j in range(cpw):
            copies.append(pltpu.async_copy(
                mt_hbm.at[midx_v.at[j]],
                rows_v.at[pl.ds(j * IDX_CHUNK, IDX_CHUNK)], sem))
        for c in copies:
            c.wait()
        pltpu.sync_copy(rows_v, m_out.at[pl.ds(base, bpw)])

    return gather_kernel(ut2, mt2, uj2, mj2)


def _pick32(x128, k):
    r = x128[:, 0:32]
    for t in (1, 2, 3):
        r = jnp.where(k == t, x128[:, 32 * t:32 * t + 32], r)
    return r


def _mlp_body(u_ref, m_ref, gao_ref, genres_ref, ctab_ref, uaux_ref, maux_ref,
              w1_ref, b1_ref, w2_ref, b2_ref, w3_ref, b3_ref, out_ref):
    blk = u_ref.shape[0]
    lanes = lax.broadcasted_iota(jnp.int32, (blk, 128), 1)
    lanes64 = lax.broadcasted_iota(jnp.int32, (blk, 64), 1)
    gao = gao_ref[...]
    g = gao[:, 0:1]
    a = gao[:, 1:2]
    o = gao[:, 2:3]
    uk = gao[:, 3:4]
    mk = gao[:, 4:5]
    uid = gao[:, 5:6]
    mid = gao[:, 6:7]
    u = _pick32(u_ref[...], uk)
    m = _pick32(m_ref[...], mk)
    uoh = (lanes64 == (uid - U_FULL)).astype(jnp.float32)
    moh = (lanes64 == (mid - M_FULL)).astype(jnp.float32)
    uaux = jnp.dot(uoh, uaux_ref[...], preferred_element_type=jnp.float32)
    maux = jnp.dot(moh, maux_ref[...], preferred_element_type=jnp.float32)
    u = jnp.where(uid >= U_FULL, uaux, u)
    m = jnp.where(mid >= M_FULL, maux, m)
    oh = ((lanes == g) | (lanes == (a + 2)) | (lanes == (o + 12)))
    demo = jnp.dot(oh.astype(jnp.float32), ctab_ref[...],
                   preferred_element_type=jnp.float32)
    h1 = (
        jnp.dot(u, w1_ref[0:32, :], preferred_element_type=jnp.float32)
        + jnp.dot(m, w1_ref[32:64, :], preferred_element_type=jnp.float32)
        + jnp.dot(demo, w1_ref[64:80, :], preferred_element_type=jnp.float32)
        + jnp.dot(genres_ref[...], w1_ref[80:98, :], preferred_element_type=jnp.float32)
        + b1_ref[...]
    )
    h1 = jnp.maximum(h1, 0.0)
    h2 = jnp.maximum(
        jnp.dot(h1, w2_ref[...], preferred_element_type=jnp.float32) + b2_ref[...],
        0.0)
    out_ref[...] = (jnp.dot(h2, w3_ref[...], preferred_element_type=jnp.float32)
                    + b3_ref[...])


def _tc_mlp(u128, m128, gao, genres, ctab, uaux, maux,
            W1, b1, W2, b2, W3, b3, blk=2048):
    grid = B // blk
    full = lambda i: (0, 0)
    return pl.pallas_call(
        _mlp_body,
        grid=(grid,),
        in_specs=[
            pl.BlockSpec((blk, 128), lambda i: (i, 0)),
            pl.BlockSpec((blk, 128), lambda i: (i, 0)),
            pl.BlockSpec((blk, 8), lambda i: (i, 0)),
            pl.BlockSpec((blk, 18), lambda i: (i, 0)),
            pl.BlockSpec((128, 16), full),
            pl.BlockSpec((64, 32), full),
            pl.BlockSpec((64, 32), full),
            pl.BlockSpec((98, 128), full),
            pl.BlockSpec((1, 128), full),
            pl.BlockSpec((128, 64), full),
            pl.BlockSpec((1, 64), full),
            pl.BlockSpec((64, 1), full),
            pl.BlockSpec((1, 1), full),
        ],
        out_specs=pl.BlockSpec((blk, 1), lambda i: (i, 0)),
        out_shape=jax.ShapeDtypeStruct((B, 1), jnp.float32),
    )(u128, m128, gao, genres, ctab, uaux, maux, W1, b1, W2, b2, W3, b3)


def kernel(user, movie, gender, age, occupation, genres,
           user_table, movie_table, gender_table, age_table, occ_table,
           W1, b1, W2, b2, W3, b3):
    user = user.astype(jnp.int32)
    movie = movie.astype(jnp.int32)
    up, mp = _sc_transpose_pack(user_table.T, movie_table.T)
    uj2 = (user // 4).reshape(B // IDX_CHUNK, IDX_CHUNK)
    mj2 = (movie // 4).reshape(B // IDX_CHUNK, IDX_CHUNK)
    u128, m128 = _sc_gather(up, mp, uj2, mj2)

    ctab = jnp.zeros((128, 16), jnp.float32)
    ctab = ctab.at[0:2, 0:4].set(gender_table)
    ctab = ctab.at[2:12, 4:8].set(age_table)
    ctab = ctab.at[12:37, 8:16].set(occ_table)

    uaux = user_table[U_FULL:]
    maux = jnp.zeros((64, EMB), jnp.float32).at[0:NM - M_FULL].set(
        movie_table[M_FULL:])

    gao = jnp.stack([gender.astype(jnp.int32), age.astype(jnp.int32),
                     occupation.astype(jnp.int32),
                     user % 4, movie % 4, user, movie,
                     jnp.zeros((B,), jnp.int32)], axis=1)

    out = _tc_mlp(u128, m128, gao, genres, ctab, uaux, maux,
                  W1, b1.reshape(1, 128), W2, b2.reshape(1, 64),
                  W3, b3.reshape(1, 1))
    return jnp.squeeze(out, axis=1)
